# baseline (speedup 1.0000x reference)
import functools

import jax
import jax.numpy as jnp
from jax import lax
from jax.experimental import pallas as pl
from jax.experimental.pallas import tpu as pltpu
from jax.experimental.pallas import tpu_sc as plsc

_NC = 2
_NS = 16
_L = 16
_NW = _NC * _NS

_CH = 80
_RB = 4
_RI = 8


def _lin_body(x_ref, w_ref, asv_ref, adv_ref,
              xs_ref, as_ref, ad_ref, ms_ref, md_ref):
    i = pl.program_id(0)
    xs = jnp.dot(x_ref[...], w_ref[...], preferred_element_type=jnp.float32)
    xs_ref[...] = xs
    asb = jnp.sum(xs * asv_ref[...], axis=1, keepdims=True)
    adb = jnp.sum(xs * adv_ref[...], axis=1, keepdims=True)
    as_ref[...] = asb
    ad_ref[...] = adb
    mxs = jnp.max(asb, keepdims=True)
    mxd = jnp.max(adb, keepdims=True)

    @pl.when(i == 0)
    def _():
        ms_ref[...] = mxs
        md_ref[...] = mxd

    @pl.when(i > 0)
    def _():
        ms_ref[...] = jnp.maximum(ms_ref[...], mxs)
        md_ref[...] = jnp.maximum(md_ref[...], mxd)


def _out_body(xs_ref, a0_ref, a1_ref, d0_ref, d1_ref, b_ref, o_ref):
    den = d0_ref[0] + d1_ref[0] + 1e-16
    o_ref[...] = xs_ref[...] + (a0_ref[0] + a1_ref[0]) / den + b_ref[...]


def _make_sc_kernel(n, c, e):
    chunks_total = e // _CH
    cpt_small = (chunks_total // _NW) // _RI * _RI
    cpt_big = cpt_small + _RI
    n_big = (chunks_total - cpt_small * _NW) // _RI
    rpt = 640
    rpt_last = n - rpt * (_NS - 1)
    den_chunks = n // _CH

    mesh = plsc.VectorSubcoreMesh(core_axis_name="c", subcore_axis_name="s")

    @functools.partial(
        pl.kernel,
        out_type=(
            jax.ShapeDtypeStruct((_NC, n, c), jnp.float32),
            jax.ShapeDtypeStruct((_NC, n), jnp.float32),
        ),
        mesh=mesh,
        compiler_params=pltpu.CompilerParams(needs_layout_passes=False),
        scratch_types=(
            [
                pltpu.VMEM((_L,), jnp.float32),
                pltpu.VMEM((_RI, _CH), jnp.int32),
                pltpu.VMEM((_RI, _CH), jnp.int32),
                pltpu.VMEM((_RB, _CH), jnp.float32),
                pltpu.VMEM((_RB, _CH), jnp.float32),
                pltpu.VMEM((_RB, _CH), jnp.float32),
                pltpu.VMEM((_RB, _CH, c), jnp.float32),
                pltpu.VMEM_SHARED((n, c), jnp.float32),
                pltpu.VMEM_SHARED((n,), jnp.float32),
            ]
            + [pltpu.SemaphoreType.DMA] * (4 * _RB + _RI)
        ),
    )
    def sc_kernel(asrc_hbm, adst_hbm, src_hbm, dst_hbm, xs_hbm, shift_hbm,
                  aggp_hbm, denp_hbm,
                  shiftb, srcr, dstr, asg, adg, exr, rows, agg_sh, den_sh,
                  *sems):
        gsem = sems[0:_RB]
        asem = sems[_RB:2 * _RB]
        ssem = sems[2 * _RB:3 * _RB]
        esem = sems[3 * _RB:4 * _RB]
        isem = sems[4 * _RB:4 * _RB + _RI]

        cid = lax.axis_index("c")
        sid = lax.axis_index("s")
        wid = cid * _NS + sid
        big = wid < n_big
        n_chunks = jnp.where(big, cpt_big, cpt_small)
        ebase = jnp.where(
            big, wid * cpt_big * _CH,
            (n_big * cpt_big + (wid - n_big) * cpt_small) * _CH)

        pltpu.sync_copy(shift_hbm, shiftb)

        def idx_copies(k, b8):
            off = ebase + k * _CH
            return (
                pltpu.make_async_copy(src_hbm.at[pl.ds(off, _CH)],
                                      srcr.at[b8], isem[b8]),
                pltpu.make_async_copy(dst_hbm.at[pl.ds(off, _CH)],
                                      dstr.at[b8], isem[b8]),
            )

        def alpha_copies(b4, b8):
            return (
                pltpu.make_async_copy(asrc_hbm.at[srcr.at[b8]],
                                      asg.at[b4], asem[b4]),
                pltpu.make_async_copy(adst_hbm.at[dstr.at[b8]],
                                      adg.at[b4], asem[b4]),
            )

        def rows_copy(b4, b8):
            return pltpu.make_async_copy(xs_hbm.at[srcr.at[b8]],
                                         rows.at[b4], gsem[b4])

        def agg_scatter(b4, b8):
            return pltpu.async_copy(rows.at[b4], agg_sh.at[dstr.at[b8]],
                                    ssem[b4], add=True)

        def agg_scatter_wait(b4, b8):
            pltpu.make_async_copy(rows.at[b4], agg_sh.at[dstr.at[b8]],
                                  ssem[b4]).wait()

        def den_scatter(b4, b8):
            return pltpu.async_copy(exr.at[b4], den_sh.at[dstr.at[b8]],
                                    esem[b4], add=True)

        def den_scatter_wait(b4, b8):
            pltpu.make_async_copy(exr.at[b4], den_sh.at[dstr.at[b8]],
                                  esem[b4]).wait()

        zv = jnp.zeros((_L,), jnp.float32)

        def _zero_rows(i, _):
            rows[0, i // 8, pl.ds((i % 8) * _L, _L)] = zv
            return 0

        lax.fori_loop(0, _CH * 8, _zero_rows, 0)

        for g in range(_CH // _L):
            asg[0, pl.ds(g * _L, _L)] = zv

        base = sid * rpt

        @pl.when(sid < _NS - 1)
        def _():
            for k in range(rpt // _CH):
                pltpu.sync_copy(rows.at[0],
                                agg_sh.at[pl.ds(base + k * _CH, _CH)])

        @pl.when(sid == _NS - 1)
        def _():
            for k in range(rpt_last // _CH):
                pltpu.sync_copy(rows.at[0],
                                agg_sh.at[pl.ds(base + k * _CH, _CH)])

        def _zero_den(j, _):
            pltpu.sync_copy(asg.at[0],
                            den_sh.at[pl.ds((sid + _NS * j) * _CH, _CH)])
            return 0

        lax.fori_loop(0, (den_chunks - sid + _NS - 1) // _NS, _zero_den, 0)

        for k0 in range(3):
            for d in idx_copies(k0, k0):
                d.start()
        for k0 in range(2):
            for d in idx_copies(k0, k0):
                d.wait()
            for d in alpha_copies(k0, k0):
                d.start()
            rows_copy(k0, k0).start()

        plsc.subcore_barrier()

        shift = shiftb[pl.ds(0, _L)]

        def _iter(i, _):
            for b in range(_RI):
                k = i * _RI + b
                b4 = b % _RB

                @pl.when(k >= _RB)
                def _():
                    den_scatter_wait(b4, (b - _RB) % _RI)

                for d2 in alpha_copies(b4, b):
                    d2.wait()

                for g in range(_CH // _L):
                    sl = pl.ds(g * _L, _L)
                    av = asg[b4, sl] + adg[b4, sl]
                    av = jnp.where(av >= 0.0, av, av * 0.2)
                    exr[b4, sl] = jnp.exp(av - shift)

                rows_copy(b4, b).wait()

                @plsc.parallel_loop(0, _CH, step=1, unroll=8)
                def _edge(t):
                    ev = plsc.load_gather(
                        exr.at[b4],
                        [jnp.broadcast_to(t, (_L,)).astype(jnp.int32)])
                    for c8 in range(c // _L):
                        sl2 = pl.ds(c8 * _L, _L)
                        rows[b4, t, sl2] = rows[b4, t, sl2] * ev

                agg_scatter(b4, b)
                den_scatter(b4, b)

                b42 = (b + 2) % _RB
                b82 = (b + 2) % _RI

                @pl.when(k + 2 < n_chunks)
                def _():
                    @pl.when(k >= 2)
                    def _():
                        agg_scatter_wait(b42, (b - 2) % _RI)

                    for d2 in idx_copies(k + 2, b82):
                        d2.wait()
                    for d2 in alpha_copies(b42, b82):
                        d2.start()
                    rows_copy(b42, b82).start()

                @pl.when(k + 3 < n_chunks)
                def _():
                    for d2 in idx_copies(k + 3, (b + 3) % _RI):
                        d2.start()
            return 0

        lax.fori_loop(0, n_chunks // _RI, _iter, 0)

        for j in range(_RB):
            agg_scatter_wait(j, _RI - _RB + j)
            den_scatter_wait(j, _RI - _RB + j)

        plsc.subcore_barrier()

        @pl.when(sid < _NS - 1)
        def _():
            pltpu.sync_copy(agg_sh.at[pl.ds(base, rpt)],
                            aggp_hbm.at[cid, pl.ds(base, rpt)])

        @pl.when(sid == _NS - 1)
        def _():
            pltpu.sync_copy(agg_sh.at[pl.ds(base, rpt_last)],
                            aggp_hbm.at[cid, pl.ds(base, rpt_last)])

        @pl.when(sid == 0)
        def _():
            pltpu.sync_copy(den_sh, denp_hbm.at[cid])

    return sc_kernel


def kernel(x, edge_index, W, att_src, att_dst, bias):
    n, d = x.shape
    c = W.shape[1]
    e = edge_index.shape[1]

    rows_blk = 2000
    grid = (n // rows_blk,)

    asv = att_src.reshape(1, c)
    adv = att_dst.reshape(1, c)

    xs, a_s, a_d, ms, md = pl.pallas_call(
        _lin_body,
        grid=grid,
        in_specs=[
            pl.BlockSpec((rows_blk, d), lambda i: (i, 0)),
            pl.BlockSpec((d, c), lambda i: (0, 0)),
            pl.BlockSpec((1, c), lambda i: (0, 0)),
            pl.BlockSpec((1, c), lambda i: (0, 0)),
        ],
        out_specs=[
            pl.BlockSpec((rows_blk, c), lambda i: (i, 0)),
            pl.BlockSpec((rows_blk, 1), lambda i: (i, 0)),
            pl.BlockSpec((rows_blk, 1), lambda i: (i, 0)),
            pl.BlockSpec((1, 1), lambda i: (0, 0)),
            pl.BlockSpec((1, 1), lambda i: (0, 0)),
        ],
        out_shape=[
            jax.ShapeDtypeStruct((n, c), jnp.float32),
            jax.ShapeDtypeStruct((n, 1), jnp.float32),
            jax.ShapeDtypeStruct((n, 1), jnp.float32),
            jax.ShapeDtypeStruct((1, 1), jnp.float32),
            jax.ShapeDtypeStruct((1, 1), jnp.float32),
        ],
    )(x, W, asv, adv)

    sh = ms[0, 0] + md[0, 0]
    sh = jnp.where(sh >= 0.0, sh, sh * 0.2)
    shift_arr = jnp.broadcast_to(sh, (_L,))

    src = edge_index[0].astype(jnp.int32)
    dst = edge_index[1].astype(jnp.int32)

    sc_k = _make_sc_kernel(n, c, e)
    aggp, denp = sc_k(a_s.reshape(n), a_d.reshape(n), src, dst, xs,
                      shift_arr)

    out = pl.pallas_call(
        _out_body,
        grid=grid,
        in_specs=[
            pl.BlockSpec((rows_blk, c), lambda i: (i, 0)),
            pl.BlockSpec((1, rows_blk, c), lambda i: (0, i, 0)),
            pl.BlockSpec((1, rows_blk, c), lambda i: (1, i, 0)),
            pl.BlockSpec((1, rows_blk, 1), lambda i: (0, i, 0)),
            pl.BlockSpec((1, rows_blk, 1), lambda i: (1, i, 0)),
            pl.BlockSpec((1, c), lambda i: (0, 0)),
        ],
        out_specs=pl.BlockSpec((rows_blk, c), lambda i: (i, 0)),
        out_shape=jax.ShapeDtypeStruct((n, c), jnp.float32),
    )(xs, aggp, aggp, denp.reshape(_NC, n, 1), denp.reshape(_NC, n, 1),
      bias.reshape(1, c))
    return out


# speedup vs baseline: 1.0670x; 1.0670x over previous
import functools

import jax
import jax.numpy as jnp
from jax import lax
from jax.experimental import pallas as pl
from jax.experimental.pallas import tpu as pltpu
from jax.experimental.pallas import tpu_sc as plsc

_NC = 2
_NS = 16
_L = 16
_NW = _NC * _NS

_CH = 80
_RB = 4
_RI = 8


def _lin_body(x_ref, w_ref, asv_ref, adv_ref,
              xs_ref, as_ref, ad_ref, mm_ref, sh_ref):
    i = pl.program_id(0)
    ng = pl.num_programs(0)
    xs = jnp.dot(x_ref[...], w_ref[...], preferred_element_type=jnp.float32)
    xs_ref[...] = xs
    asb = jnp.sum(xs * asv_ref[...], axis=1, keepdims=True)
    adb = jnp.sum(xs * adv_ref[...], axis=1, keepdims=True)
    as_ref[...] = asb
    ad_ref[...] = adb
    mx = jnp.concatenate(
        [jnp.max(asb, keepdims=True), jnp.max(adb, keepdims=True)], axis=1)

    @pl.when(i == 0)
    def _():
        mm_ref[...] = mx

    @pl.when(i > 0)
    def _():
        mm_ref[...] = jnp.maximum(mm_ref[...], mx)

    @pl.when(i == ng - 1)
    def _():
        sh = jnp.sum(mm_ref[...])
        sh = jnp.where(sh >= 0.0, sh, sh * 0.2)
        sh_ref[...] = jnp.broadcast_to(sh, (1, _L))


def _out_body(xs_ref, a0_ref, a1_ref, d0_ref, d1_ref, b_ref, o_ref):
    den = d0_ref[0] + d1_ref[0] + 1e-16
    o_ref[...] = xs_ref[...] + (a0_ref[0] + a1_ref[0]) / den + b_ref[...]


def _make_sc_kernel(n, c, e):
    chunks_total = e // _CH
    cpt_small = (chunks_total // _NW) // _RI * _RI
    cpt_big = cpt_small + _RI
    n_big = (chunks_total - cpt_small * _NW) // _RI
    rpt = 640
    rpt_last = n - rpt * (_NS - 1)
    den_chunks = n // _CH

    mesh = plsc.VectorSubcoreMesh(core_axis_name="c", subcore_axis_name="s")

    @functools.partial(
        pl.kernel,
        out_type=(
            jax.ShapeDtypeStruct((_NC, n, c), jnp.float32),
            jax.ShapeDtypeStruct((_NC, n), jnp.float32),
        ),
        mesh=mesh,
        compiler_params=pltpu.CompilerParams(needs_layout_passes=False),
        scratch_types=(
            [
                pltpu.VMEM((_L,), jnp.float32),
                pltpu.VMEM((_RI, _CH), jnp.int32),
                pltpu.VMEM((_RI, _CH), jnp.int32),
                pltpu.VMEM((_RB, _CH), jnp.float32),
                pltpu.VMEM((_RB, _CH), jnp.float32),
                pltpu.VMEM((_RB, _CH), jnp.float32),
                pltpu.VMEM((_RB, _CH, c), jnp.float32),
                pltpu.VMEM_SHARED((n, c), jnp.float32),
                pltpu.VMEM_SHARED((n,), jnp.float32),
            ]
            + [pltpu.SemaphoreType.DMA] * (4 * _RB + _RI)
        ),
    )
    def sc_kernel(asrc_hbm, adst_hbm, ei_hbm, xs_hbm, shift_hbm,
                  aggp_hbm, denp_hbm,
                  shiftb, srcr, dstr, asg, adg, exr, rows, agg_sh, den_sh,
                  *sems):
        gsem = sems[0:_RB]
        asem = sems[_RB:2 * _RB]
        ssem = sems[2 * _RB:3 * _RB]
        esem = sems[3 * _RB:4 * _RB]
        isem = sems[4 * _RB:4 * _RB + _RI]

        cid = lax.axis_index("c")
        sid = lax.axis_index("s")
        wid = cid * _NS + sid
        big = wid < n_big
        n_chunks = jnp.where(big, cpt_big, cpt_small)
        ebase = jnp.where(
            big, wid * cpt_big * _CH,
            (n_big * cpt_big + (wid - n_big) * cpt_small) * _CH)

        pltpu.sync_copy(shift_hbm, shiftb)

        def idx_copies(k, b8):
            off = ebase + k * _CH
            return (
                pltpu.make_async_copy(ei_hbm.at[pl.ds(off, _CH)],
                                      srcr.at[b8], isem[b8]),
                pltpu.make_async_copy(ei_hbm.at[pl.ds(e + off, _CH)],
                                      dstr.at[b8], isem[b8]),
            )

        def alpha_copies(b4, b8):
            return (
                pltpu.make_async_copy(asrc_hbm.at[srcr.at[b8]],
                                      asg.at[b4], asem[b4]),
                pltpu.make_async_copy(adst_hbm.at[dstr.at[b8]],
                                      adg.at[b4], asem[b4]),
            )

        def rows_copy(b4, b8):
            return pltpu.make_async_copy(xs_hbm.at[srcr.at[b8]],
                                         rows.at[b4], gsem[b4])

        def agg_scatter(b4, b8):
            return pltpu.async_copy(rows.at[b4], agg_sh.at[dstr.at[b8]],
                                    ssem[b4], add=True)

        def agg_scatter_wait(b4, b8):
            pltpu.make_async_copy(rows.at[b4], agg_sh.at[dstr.at[b8]],
                                  ssem[b4]).wait()

        def den_scatter(b4, b8):
            return pltpu.async_copy(exr.at[b4], den_sh.at[dstr.at[b8]],
                                    esem[b4], add=True)

        def den_scatter_wait(b4, b8):
            pltpu.make_async_copy(exr.at[b4], den_sh.at[dstr.at[b8]],
                                  esem[b4]).wait()

        zv = jnp.zeros((_L,), jnp.float32)

        def _zero_rows(i, _):
            rows[0, i // 8, pl.ds((i % 8) * _L, _L)] = zv
            return 0

        lax.fori_loop(0, _CH * 8, _zero_rows, 0)

        for g in range(_CH // _L):
            asg[0, pl.ds(g * _L, _L)] = zv

        base = sid * rpt

        @pl.when(sid < _NS - 1)
        def _():
            zc = [pltpu.make_async_copy(
                rows.at[0], agg_sh.at[pl.ds(base + k * _CH, _CH)], isem[0])
                for k in range(rpt // _CH)]
            for d2 in zc:
                d2.start()
            for d2 in zc:
                d2.wait()

        @pl.when(sid == _NS - 1)
        def _():
            zc = [pltpu.make_async_copy(
                rows.at[0], agg_sh.at[pl.ds(base + k * _CH, _CH)], isem[0])
                for k in range(rpt_last // _CH)]
            for d2 in zc:
                d2.start()
            for d2 in zc:
                d2.wait()

        def _zero_den(j, _):
            pltpu.sync_copy(asg.at[0],
                            den_sh.at[pl.ds((sid + _NS * j) * _CH, _CH)])
            return 0

        lax.fori_loop(0, (den_chunks - sid + _NS - 1) // _NS, _zero_den, 0)

        for k0 in range(3):
            for d in idx_copies(k0, k0):
                d.start()
        for k0 in range(2):
            for d in idx_copies(k0, k0):
                d.wait()
            for d in alpha_copies(k0, k0):
                d.start()
            rows_copy(k0, k0).start()

        plsc.subcore_barrier()

        shift = shiftb[pl.ds(0, _L)]

        def _iter(i, _):
            for b in range(_RI):
                k = i * _RI + b
                b4 = b % _RB

                @pl.when(k >= _RB)
                def _():
                    den_scatter_wait(b4, (b - _RB) % _RI)

                for d2 in alpha_copies(b4, b):
                    d2.wait()

                for g in range(_CH // _L):
                    sl = pl.ds(g * _L, _L)
                    av = asg[b4, sl] + adg[b4, sl]
                    av = jnp.where(av >= 0.0, av, av * 0.2)
                    exr[b4, sl] = jnp.exp(av - shift)

                rows_copy(b4, b).wait()

                @plsc.parallel_loop(0, _CH, step=1, unroll=4)
                def _edge(t):
                    ev = plsc.load_gather(
                        exr.at[b4],
                        [jnp.broadcast_to(t, (_L,)).astype(jnp.int32)])
                    for c8 in range(c // _L):
                        sl2 = pl.ds(c8 * _L, _L)
                        rows[b4, t, sl2] = rows[b4, t, sl2] * ev

                agg_scatter(b4, b)
                den_scatter(b4, b)

                b42 = (b + 2) % _RB
                b82 = (b + 2) % _RI

                @pl.when(k + 2 < n_chunks)
                def _():
                    @pl.when(k >= 2)
                    def _():
                        agg_scatter_wait(b42, (b - 2) % _RI)

                    for d2 in idx_copies(k + 2, b82):
                        d2.wait()
                    for d2 in alpha_copies(b42, b82):
                        d2.start()
                    rows_copy(b42, b82).start()

                @pl.when(k + 3 < n_chunks)
                def _():
                    for d2 in idx_copies(k + 3, (b + 3) % _RI):
                        d2.start()
            return 0

        lax.fori_loop(0, n_chunks // _RI, _iter, 0)

        for j in range(_RB):
            agg_scatter_wait(j, _RI - _RB + j)
            den_scatter_wait(j, _RI - _RB + j)

        plsc.subcore_barrier()

        @pl.when(sid < _NS - 1)
        def _():
            pltpu.sync_copy(agg_sh.at[pl.ds(base, rpt)],
                            aggp_hbm.at[cid, pl.ds(base, rpt)])

        @pl.when(sid == _NS - 1)
        def _():
            pltpu.sync_copy(agg_sh.at[pl.ds(base, rpt_last)],
                            aggp_hbm.at[cid, pl.ds(base, rpt_last)])

        @pl.when(sid == 0)
        def _():
            pltpu.sync_copy(den_sh, denp_hbm.at[cid])

    return sc_kernel


def kernel(x, edge_index, W, att_src, att_dst, bias):
    n, d = x.shape
    c = W.shape[1]
    e = edge_index.shape[1]

    rows_blk = 1000
    grid = (n // rows_blk,)

    asv = att_src.reshape(1, c)
    adv = att_dst.reshape(1, c)

    xs, a_s, a_d, _, shift_arr = pl.pallas_call(
        _lin_body,
        grid=grid,
        in_specs=[
            pl.BlockSpec((rows_blk, d), lambda i: (i, 0)),
            pl.BlockSpec((d, c), lambda i: (0, 0)),
            pl.BlockSpec((1, c), lambda i: (0, 0)),
            pl.BlockSpec((1, c), lambda i: (0, 0)),
        ],
        out_specs=[
            pl.BlockSpec((rows_blk, c), lambda i: (i, 0)),
            pl.BlockSpec((rows_blk, 1), lambda i: (i, 0)),
            pl.BlockSpec((rows_blk, 1), lambda i: (i, 0)),
            pl.BlockSpec((1, 2), lambda i: (0, 0)),
            pl.BlockSpec((1, _L), lambda i: (0, 0)),
        ],
        out_shape=[
            jax.ShapeDtypeStruct((n, c), jnp.float32),
            jax.ShapeDtypeStruct((n, 1), jnp.float32),
            jax.ShapeDtypeStruct((n, 1), jnp.float32),
            jax.ShapeDtypeStruct((1, 2), jnp.float32),
            jax.ShapeDtypeStruct((1, _L), jnp.float32),
        ],
    )(x, W, asv, adv)

    sc_k = _make_sc_kernel(n, c, e)
    aggp, denp = sc_k(a_s.reshape(n), a_d.reshape(n),
                      edge_index.reshape(2 * e), xs, shift_arr.reshape(_L))

    out = pl.pallas_call(
        _out_body,
        grid=grid,
        in_specs=[
            pl.BlockSpec((rows_blk, c), lambda i: (i, 0)),
            pl.BlockSpec((1, rows_blk, c), lambda i: (0, i, 0)),
            pl.BlockSpec((1, rows_blk, c), lambda i: (1, i, 0)),
            pl.BlockSpec((1, rows_blk, 1), lambda i: (0, i, 0)),
            pl.BlockSpec((1, rows_blk, 1), lambda i: (1, i, 0)),
            pl.BlockSpec((1, c), lambda i: (0, 0)),
        ],
        out_specs=pl.BlockSpec((rows_blk, c), lambda i: (i, 0)),
        out_shape=jax.ShapeDtypeStruct((n, c), jnp.float32),
    )(xs, aggp, aggp, denp.reshape(_NC, n, 1), denp.reshape(_NC, n, 1),
      bias.reshape(1, c))
    return out
